# baseline (speedup 1.0000x reference)
import functools

import jax
import jax.numpy as jnp
from jax import lax
from jax.experimental import pallas as pl
from jax.experimental.pallas import tpu as pltpu
from jax.experimental.pallas import tpu_sc as plsc

N = 50000
NP = 50176
FP = 16
H = 256
G = H // FP
NG = 16
E = 800000
NW = 32
CHUNK = 128
CH0 = 224
CH1 = 168
EP = 16 * (CH0 + CH1) * CHUNK
NBUF = 4
RT = NP // 16
ZROWS = 256

_SELU_ALPHA = 1.6732632423543772
_SELU_SCALE = 1.0507009873554805


def _selu(x):
  return _SELU_SCALE * jnp.where(x > 0, x, _SELU_ALPHA * (jnp.exp(x) - 1.0))


def _make_prop(groups: int, scale: int, out_cols: int, width: int = FP):
  mesh = plsc.VectorSubcoreMesh(core_axis_name="c", subcore_axis_name="s")

  @functools.partial(
      pl.kernel,
      mesh=mesh,
      compiler_params=pltpu.CompilerParams(use_tc_tiling_on_sc=False),
      out_type=jax.ShapeDtypeStruct((2, NP, out_cols), jnp.float32),
      scratch_types=[
          pltpu.VMEM((NBUF, CHUNK), jnp.int32),
          pltpu.VMEM((NBUF, CHUNK), jnp.int32),
          pltpu.VMEM((NBUF, CHUNK), jnp.int32),
          pltpu.VMEM((NBUF, CHUNK, width), jnp.float32),
          pltpu.VMEM((ZROWS, width), jnp.float32),
          pltpu.VMEM_SHARED((NP, width), jnp.float32),
          pltpu.SemaphoreType.DMA((NBUF,)),
          pltpu.SemaphoreType.DMA((NBUF,)),
          pltpu.SemaphoreType.DMA((NBUF,)),
          pltpu.SemaphoreType.DMA((NBUF,)),
      ],
  )
  def prop(table, srci, dsti, out, srcring, dstring, idxg, ring, zbuf, acc,
           gsem, ssem, issem, idsem):
    c = lax.axis_index("c")
    s = lax.axis_index("s")
    base = s * RT
    nch = jnp.where(c == 0, CH0, CH1)

    zv = jnp.zeros((16,), jnp.float32)

    def zrow(i, _):
      for hh in range(width // 16):
        zbuf[i, pl.ds(hh * 16, 16)] = zv
      return ()

    lax.fori_loop(0, ZROWS, zrow, ())

    def start_idx(b, j):
      pltpu.make_async_copy(srci.at[c, s, j], srcring.at[b],
                            issem.at[b]).start()
      pltpu.make_async_copy(dsti.at[c, s, j], dstring.at[b],
                            idsem.at[b]).start()

    def wait_idx(b, j):
      pltpu.make_async_copy(srci.at[c, s, j], srcring.at[b],
                            issem.at[b]).wait()
      pltpu.make_async_copy(dsti.at[c, s, j], dstring.at[b],
                            idsem.at[b]).wait()

    def compute_idx(b, offv):
      for t in range(CHUNK // 16):
        idxg[b, pl.ds(t * 16, 16)] = (
            srcring[b, pl.ds(t * 16, 16)] * scale + offv)

    def start_gather(b):
      pltpu.make_async_copy(table.at[idxg.at[b]], ring.at[b],
                            gsem.at[b]).start()

    def wait_gather(b):
      pltpu.make_async_copy(table.at[idxg.at[b]], ring.at[b],
                            gsem.at[b]).wait()

    def gbody(g, _):
      for k in range(12):
        pltpu.sync_copy(zbuf, acc.at[pl.ds(base + k * ZROWS, ZROWS)])
      pltpu.sync_copy(zbuf.at[pl.ds(0, 64)], acc.at[pl.ds(base + 3072, 64)])
      plsc.subcore_barrier()

      offv = jnp.full((16,), g, jnp.int32)
      for b in range(NBUF):
        start_idx(b, b)

      def jbody(t, _):
        j0 = t * NBUF
        for b in range(NBUF):
          j = j0 + b
          wait_idx(b, j)
          compute_idx(b, offv)
          start_gather(b)
        for b in range(NBUF):
          wait_gather(b)
          pltpu.async_copy(ring.at[b], acc.at[dstring.at[b]], ssem.at[b],
                           add=True)
        for b in range(NBUF):
          j = j0 + b
          jn = j + NBUF
          pltpu.make_async_copy(ring.at[b], acc.at[dstring.at[b]],
                                ssem.at[b]).wait()

          @pl.when(jn < nch)
          def _():
            start_idx(b, jn)

        return ()

      lax.fori_loop(0, nch // NBUF, jbody, ())
      plsc.subcore_barrier()
      pltpu.sync_copy(acc.at[pl.ds(base, RT)],
                      out.at[c, pl.ds(base, RT), pl.ds(g * width, width)])
      return ()

    lax.fori_loop(0, groups, gbody, ())

  return prop


_prop_deg = _make_prop(1, 1, 128)
_prop_l1 = _make_prop(1, 8, 128)
_prop_l2 = _make_prop(8, 8, H, width=32)


def _prep_body(dp_ref, xp_ref, dinv_ref, xs_ref):
  deg = dp_ref[0][:, :1] + dp_ref[1][:, :1] + 1.0
  dinv = lax.rsqrt(deg)
  dinv_ref[...] = jnp.broadcast_to(dinv, dinv_ref.shape)
  xs_ref[...] = dinv * xp_ref[...]


def _mm1_body(xs_ref, p_ref, dinv_ref, w_ref, b_ref, out_ref):
  dinv = dinv_ref[...][:, :1]
  t = (dinv * (p_ref[0] + p_ref[1] + xs_ref[...]))[:, :FP]
  h = jnp.dot(t, w_ref[...], preferred_element_type=jnp.float32) + b_ref[...]
  h = _selu(h)
  hs = dinv * h
  out_ref[:, 0, :] = hs[:, :128]
  out_ref[:, 1, :] = hs[:, 128:]


def _mm2_body(q_ref, hs_ref, dinv_ref, w_ref, b_ref, bt_ref, out_ref):
  halves = [q_ref[0, :, k, :] + q_ref[1, :, k, :] + hs_ref[:, k, :]
            for k in range(2)]
  u = dinv_ref[...][:, :1] * jnp.concatenate(halves, axis=1)
  h2 = _selu(jnp.dot(u, w_ref[...], preferred_element_type=jnp.float32)
             + b_ref[...])
  bt = bt_ref[...].astype(jnp.int32)
  oh = (bt == lax.broadcasted_iota(jnp.int32, (1, NG), 1)).astype(jnp.float32)
  ext = jnp.concatenate(
      [h2, jnp.ones((h2.shape[0], 128), jnp.float32)], axis=1)
  part = lax.dot_general(oh, ext, (((0,), (0,)), ((), ())),
                         preferred_element_type=jnp.float32)

  @pl.when(pl.program_id(0) == 0)
  def _():
    out_ref[...] = jnp.zeros_like(out_ref)

  out_ref[...] += part


def _head_body(pool_ref, wf1_ref, bf1_ref, wf2_ref, bf2_ref, out_ref):
  seg = pool_ref[:, :H]
  cnt = pool_ref[:, H:H + 1]
  pooled = seg / jnp.maximum(cnt, 1.0)
  p = _selu(pooled)
  a = _selu(jnp.dot(p, wf1_ref[...], preferred_element_type=jnp.float32)
            + bf1_ref[...])
  o = jnp.dot(a, wf2_ref[...], preferred_element_type=jnp.float32) \
      + bf2_ref[...]
  m = jnp.max(o, axis=1, keepdims=True)
  e = o - m
  lse = jnp.log(jnp.sum(jnp.exp(e), axis=1, keepdims=True))
  out_ref[...] = e - lse


def kernel(x, edge_index, batch, W1, b1, W2, b2, Wf1, bf1, Wf2, bf2):
  f32 = jnp.float32
  src = edge_index[0].astype(jnp.int32)
  dst = edge_index[1].astype(jnp.int32)

  def _split(a):
    n0 = 16 * CH0 * CHUNK
    a0 = a[:n0].reshape(16, CH0, CHUNK)
    a1 = a[n0:].reshape(16, CH1, CHUNK)
    a1 = jnp.pad(a1, ((0, 0), (0, CH0 - CH1), (0, 0)))
    return jnp.stack([a0, a1])

  srcp = _split(jnp.concatenate([src, jnp.zeros((EP - E,), jnp.int32)]))
  dstp = _split(jnp.concatenate([dst, jnp.full((EP - E,), N, jnp.int32)]))

  xpad = jnp.zeros((NP, 128), f32).at[:N, :x.shape[1]].set(x)
  batchp = jnp.concatenate(
      [batch.astype(jnp.int8), jnp.full((NP - N,), NG, jnp.int8)])
  batchp = batchp.reshape(NP, 1)
  ones_tab = jnp.ones((NP, FP), f32)
  W1p = jnp.zeros((FP, H), f32).at[:W1.shape[0]].set(W1)

  dparts = _prop_deg(ones_tab, srcp, dstp)

  nb = NP // 512
  dinv, xs = pl.pallas_call(
      _prep_body,
      grid=(nb,),
      in_specs=[
          pl.BlockSpec((2, 512, 128), lambda i: (0, i, 0)),
          pl.BlockSpec((512, 128), lambda i: (i, 0)),
      ],
      out_specs=[
          pl.BlockSpec((512, 128), lambda i: (i, 0)),
          pl.BlockSpec((512, 128), lambda i: (i, 0)),
      ],
      out_shape=[
          jax.ShapeDtypeStruct((NP, 128), f32),
          jax.ShapeDtypeStruct((NP, 128), f32),
      ],
  )(dparts, xpad)

  p1 = _prop_l1(xs.reshape(NP * 8, FP), srcp, dstp)

  hs2 = pl.pallas_call(
      _mm1_body,
      grid=(nb,),
      in_specs=[
          pl.BlockSpec((512, 128), lambda i: (i, 0)),
          pl.BlockSpec((2, 512, 128), lambda i: (0, i, 0)),
          pl.BlockSpec((512, 128), lambda i: (i, 0)),
          pl.BlockSpec((FP, H), lambda i: (0, 0)),
          pl.BlockSpec((1, H), lambda i: (0, 0)),
      ],
      out_specs=pl.BlockSpec((512, 2, 128), lambda i: (i, 0, 0)),
      out_shape=jax.ShapeDtypeStruct((NP, 2, 128), f32),
  )(xs, p1, dinv, W1p, b1.reshape(1, H))

  qparts = _prop_l2(hs2.reshape(NP * 8, 32), srcp, dstp)

  pool = pl.pallas_call(
      _mm2_body,
      grid=(nb,),
      in_specs=[
          pl.BlockSpec((2, 512, 2, 128), lambda i: (0, i, 0, 0)),
          pl.BlockSpec((512, 2, 128), lambda i: (i, 0, 0)),
          pl.BlockSpec((512, 128), lambda i: (i, 0)),
          pl.BlockSpec((H, H), lambda i: (0, 0)),
          pl.BlockSpec((1, H), lambda i: (0, 0)),
          pl.BlockSpec((512, 1), lambda i: (i, 0)),
      ],
      out_specs=pl.BlockSpec((NG, H + 128), lambda i: (0, 0)),
      out_shape=jax.ShapeDtypeStruct((NG, H + 128), f32),
  )(qparts.reshape(2, NP, 2, 128), hs2, dinv, W2, b2.reshape(1, H), batchp)

  out = pl.pallas_call(
      _head_body,
      out_shape=jax.ShapeDtypeStruct((NG, 2), f32),
  )(pool, Wf1, bf1.reshape(1, -1), Wf2, bf2.reshape(1, -1))
  return out


# speedup vs baseline: 1.0070x; 1.0070x over previous
import functools

import jax
import jax.numpy as jnp
from jax import lax
from jax.experimental import pallas as pl
from jax.experimental.pallas import tpu as pltpu
from jax.experimental.pallas import tpu_sc as plsc

N = 50000
NP = 50176
FP = 16
H = 256
G = H // FP
NG = 16
E = 800000
NW = 32
CHUNK = 128
CH0 = 224
CH1 = 168
EP = 16 * (CH0 + CH1) * CHUNK
NBUF = 4
RT = NP // 16
ZROWS = 256

_SELU_ALPHA = 1.6732632423543772
_SELU_SCALE = 1.0507009873554805


def _selu(x):
  return _SELU_SCALE * jnp.where(x > 0, x, _SELU_ALPHA * (jnp.exp(x) - 1.0))


def _make_prop(groups: int, out_cols: int, width: int = FP):
  mesh = plsc.VectorSubcoreMesh(core_axis_name="c", subcore_axis_name="s")

  @functools.partial(
      pl.kernel,
      mesh=mesh,
      compiler_params=pltpu.CompilerParams(use_tc_tiling_on_sc=False),
      out_type=jax.ShapeDtypeStruct((2, NP, out_cols), jnp.float32),
      scratch_types=[
          pltpu.VMEM((NBUF, CHUNK), jnp.int32),
          pltpu.VMEM((NBUF, CHUNK), jnp.int32),
          pltpu.VMEM((NBUF, CHUNK, width), jnp.float32),
          pltpu.VMEM((ZROWS, width), jnp.float32),
          pltpu.VMEM_SHARED((NP, width), jnp.float32),
          pltpu.SemaphoreType.DMA((NBUF,)),
          pltpu.SemaphoreType.DMA((NBUF,)),
          pltpu.SemaphoreType.DMA((NBUF,)),
          pltpu.SemaphoreType.DMA((NBUF,)),
      ],
  )
  def prop(table, srci, dsti, out, srcring, dstring, ring, zbuf, acc,
           gsem, ssem, issem, idsem):
    c = lax.axis_index("c")
    s = lax.axis_index("s")
    base = s * RT
    nch = jnp.where(c == 0, CH0, CH1)

    zv = jnp.zeros((16,), jnp.float32)

    def zrow(i, _):
      for hh in range(width // 16):
        zbuf[i, pl.ds(hh * 16, 16)] = zv
      return ()

    lax.fori_loop(0, ZROWS, zrow, ())
    plsc.subcore_barrier()

    def start_idx(b, g, j):
      pltpu.make_async_copy(srci.at[g, c, s, j], srcring.at[b],
                            issem.at[b]).start()
      pltpu.make_async_copy(dsti.at[c, s, j], dstring.at[b],
                            idsem.at[b]).start()

    def wait_idx(b, g, j):
      pltpu.make_async_copy(srci.at[g, c, s, j], srcring.at[b],
                            issem.at[b]).wait()
      pltpu.make_async_copy(dsti.at[c, s, j], dstring.at[b],
                            idsem.at[b]).wait()

    def start_gather(b):
      pltpu.make_async_copy(table.at[srcring.at[b]], ring.at[b],
                            gsem.at[b]).start()

    def wait_gather(b):
      pltpu.make_async_copy(table.at[srcring.at[b]], ring.at[b],
                            gsem.at[b]).wait()

    def gbody(g, _):
      for k in range(12):
        pltpu.sync_copy(zbuf, acc.at[pl.ds(base + k * ZROWS, ZROWS)])
      pltpu.sync_copy(zbuf.at[pl.ds(0, 64)], acc.at[pl.ds(base + 3072, 64)])
      plsc.subcore_barrier()

      for b in range(NBUF):
        start_idx(b, g, b)

      def jbody(t, _):
        j0 = t * NBUF
        for b in range(NBUF):
          j = j0 + b
          wait_idx(b, g, j)
          start_gather(b)
        for b in range(NBUF):
          wait_gather(b)
          pltpu.async_copy(ring.at[b], acc.at[dstring.at[b]], ssem.at[b],
                           add=True)
        for b in range(NBUF):
          j = j0 + b
          jn = j + NBUF
          pltpu.make_async_copy(ring.at[b], acc.at[dstring.at[b]],
                                ssem.at[b]).wait()

          @pl.when(jn < nch)
          def _():
            start_idx(b, g, jn)

        return ()

      lax.fori_loop(0, nch // NBUF, jbody, ())
      plsc.subcore_barrier()
      pltpu.sync_copy(acc.at[pl.ds(base, RT)],
                      out.at[c, pl.ds(base, RT), pl.ds(g * width, width)])
      return ()

    lax.fori_loop(0, groups, gbody, ())

  return prop


_prop_deg = _make_prop(1, 128)
_prop_l1 = _make_prop(1, 128)
_prop_l2 = _make_prop(8, H, width=32)


def _prep_body(dp_ref, xp_ref, dinv_ref, xs_ref):
  deg = dp_ref[0][:, :1] + dp_ref[1][:, :1] + 1.0
  dinv = lax.rsqrt(deg)
  dinv_ref[...] = jnp.broadcast_to(dinv, dinv_ref.shape)
  xs_ref[...] = dinv * xp_ref[...]


def _mm1_body(xs_ref, p_ref, dinv_ref, w_ref, b_ref, out_ref):
  dinv = dinv_ref[...][:, :1]
  t = (dinv * (p_ref[0] + p_ref[1] + xs_ref[...]))[:, :FP]
  h = jnp.dot(t, w_ref[...], preferred_element_type=jnp.float32) + b_ref[...]
  h = _selu(h)
  hs = dinv * h
  out_ref[:, 0, :] = hs[:, :128]
  out_ref[:, 1, :] = hs[:, 128:]


def _mm2_body(q_ref, hs_ref, dinv_ref, w_ref, b_ref, bt_ref, out_ref):
  halves = [q_ref[0, :, k, :] + q_ref[1, :, k, :] + hs_ref[:, k, :]
            for k in range(2)]
  u = dinv_ref[...][:, :1] * jnp.concatenate(halves, axis=1)
  h2 = _selu(jnp.dot(u, w_ref[...], preferred_element_type=jnp.float32)
             + b_ref[...])
  bt = bt_ref[...].astype(jnp.int32)
  oh = (bt == lax.broadcasted_iota(jnp.int32, (1, NG), 1)).astype(jnp.float32)
  ext = jnp.concatenate(
      [h2, jnp.ones((h2.shape[0], 128), jnp.float32)], axis=1)
  part = lax.dot_general(oh, ext, (((0,), (0,)), ((), ())),
                         preferred_element_type=jnp.float32)

  @pl.when(pl.program_id(0) == 0)
  def _():
    out_ref[...] = jnp.zeros_like(out_ref)

  out_ref[...] += part


def _head_body(pool_ref, wf1_ref, bf1_ref, wf2_ref, bf2_ref, out_ref):
  seg = pool_ref[:, :H]
  cnt = pool_ref[:, H:H + 1]
  pooled = seg / jnp.maximum(cnt, 1.0)
  p = _selu(pooled)
  a = _selu(jnp.dot(p, wf1_ref[...], preferred_element_type=jnp.float32)
            + bf1_ref[...])
  o = jnp.dot(a, wf2_ref[...], preferred_element_type=jnp.float32) \
      + bf2_ref[...]
  m = jnp.max(o, axis=1, keepdims=True)
  e = o - m
  lse = jnp.log(jnp.sum(jnp.exp(e), axis=1, keepdims=True))
  out_ref[...] = e - lse


def kernel(x, edge_index, batch, W1, b1, W2, b2, Wf1, bf1, Wf2, bf2):
  f32 = jnp.float32
  src = edge_index[0].astype(jnp.int32)
  dst = edge_index[1].astype(jnp.int32)

  def _split(a):
    n0 = 16 * CH0 * CHUNK
    a0 = a[:n0].reshape(16, CH0, CHUNK)
    a1 = a[n0:].reshape(16, CH1, CHUNK)
    a1 = jnp.pad(a1, ((0, 0), (0, CH0 - CH1), (0, 0)))
    return jnp.stack([a0, a1])

  srcp = _split(jnp.concatenate([src, jnp.zeros((EP - E,), jnp.int32)]))
  dstp = _split(jnp.concatenate([dst, jnp.full((EP - E,), N, jnp.int32)]))
  srcp1 = srcp[None]
  srcp8 = (srcp * 8)[None]
  srcp8g = srcp8 + jnp.arange(8, dtype=jnp.int32).reshape(8, 1, 1, 1, 1)

  xpad = jnp.zeros((NP, 128), f32).at[:N, :x.shape[1]].set(x)
  batchp = jnp.concatenate(
      [batch.astype(jnp.int8), jnp.full((NP - N,), NG, jnp.int8)])
  batchp = batchp.reshape(NP, 1)
  ones_tab = jnp.ones((NP, FP), f32)
  W1p = jnp.zeros((FP, H), f32).at[:W1.shape[0]].set(W1)

  dparts = _prop_deg(ones_tab, srcp1, dstp)

  nb = NP // 512
  dinv, xs = pl.pallas_call(
      _prep_body,
      grid=(nb,),
      in_specs=[
          pl.BlockSpec((2, 512, 128), lambda i: (0, i, 0)),
          pl.BlockSpec((512, 128), lambda i: (i, 0)),
      ],
      out_specs=[
          pl.BlockSpec((512, 128), lambda i: (i, 0)),
          pl.BlockSpec((512, 128), lambda i: (i, 0)),
      ],
      out_shape=[
          jax.ShapeDtypeStruct((NP, 128), f32),
          jax.ShapeDtypeStruct((NP, 128), f32),
      ],
  )(dparts, xpad)

  p1 = _prop_l1(xs.reshape(NP * 8, FP), srcp8, dstp)

  hs2 = pl.pallas_call(
      _mm1_body,
      grid=(nb,),
      in_specs=[
          pl.BlockSpec((512, 128), lambda i: (i, 0)),
          pl.BlockSpec((2, 512, 128), lambda i: (0, i, 0)),
          pl.BlockSpec((512, 128), lambda i: (i, 0)),
          pl.BlockSpec((FP, H), lambda i: (0, 0)),
          pl.BlockSpec((1, H), lambda i: (0, 0)),
      ],
      out_specs=pl.BlockSpec((512, 2, 128), lambda i: (i, 0, 0)),
      out_shape=jax.ShapeDtypeStruct((NP, 2, 128), f32),
  )(xs, p1, dinv, W1p, b1.reshape(1, H))

  qparts = _prop_l2(hs2.reshape(NP * 8, 32), srcp8g, dstp)

  pool = pl.pallas_call(
      _mm2_body,
      grid=(nb,),
      in_specs=[
          pl.BlockSpec((2, 512, 2, 128), lambda i: (0, i, 0, 0)),
          pl.BlockSpec((512, 2, 128), lambda i: (i, 0, 0)),
          pl.BlockSpec((512, 128), lambda i: (i, 0)),
          pl.BlockSpec((H, H), lambda i: (0, 0)),
          pl.BlockSpec((1, H), lambda i: (0, 0)),
          pl.BlockSpec((512, 1), lambda i: (i, 0)),
      ],
      out_specs=pl.BlockSpec((NG, H + 128), lambda i: (0, 0)),
      out_shape=jax.ShapeDtypeStruct((NG, H + 128), f32),
  )(qparts.reshape(2, NP, 2, 128), hs2, dinv, W2, b2.reshape(1, H), batchp)

  out = pl.pallas_call(
      _head_body,
      out_shape=jax.ShapeDtypeStruct((NG, 2), f32),
  )(pool, Wf1, bf1.reshape(1, -1), Wf2, bf2.reshape(1, -1))
  return out
